# baseline (speedup 1.0000x reference)
import functools

import jax
import jax.numpy as jnp
from jax import lax
from jax.experimental import pallas as pl
from jax.experimental.pallas import tpu as pltpu
from jax.experimental.pallas import tpu_sc as plsc

M_MAX = 8
PHI = 1.6
JITTER = 1e-4


def _gate_mlp_kernel(s_ref, mask_ref, emb_ref, ln1g_ref, ln1b_ref,
                     w1_ref, b1_ref, w2_ref, b2_ref,
                     s_child_ref, g_ref, *, kb, m):
    s_blk = s_ref[...]
    C = s_blk.shape[1]
    rows = kb * m

    s_rep = jnp.broadcast_to(s_blk[:, None, :], (kb, m, C)).reshape(rows, C)
    e_rep = jnp.broadcast_to(emb_ref[...][None, :, :], (kb, m, C)).reshape(rows, C)

    gate_in = s_rep + e_rep
    mu = jnp.mean(gate_in, axis=-1, keepdims=True)
    var = jnp.mean(jnp.square(gate_in - mu), axis=-1, keepdims=True)
    h = (gate_in - mu) * lax.rsqrt(var + 1e-5)
    h = h * ln1g_ref[...] + ln1b_ref[...]

    h1 = jnp.dot(h, w1_ref[...], preferred_element_type=jnp.float32) + b1_ref[...]
    h1 = h1 * jax.nn.sigmoid(h1)
    bg = jnp.dot(h1, w2_ref[...], preferred_element_type=jnp.float32) + b2_ref[...]

    m_rep = jnp.broadcast_to(mask_ref[...][:, None, :], (kb, m, 1)).reshape(rows, 1)
    g = jax.nn.sigmoid(bg) * m_rep

    s_child_ref[...] = g * s_rep
    g_ref[...] = jnp.transpose(g, (1, 0))[None]


def _tc_call(s2, mask2, emb, ln1g, ln1b, W1, b1, W2, b2, *, N, C, M, KB):
    rows = KB * M
    NBLK = N // KB
    kfn = functools.partial(_gate_mlp_kernel, kb=KB, m=M)
    return pl.pallas_call(
        kfn,
        grid=(NBLK,),
        in_specs=[
            pl.BlockSpec((KB, C), lambda i: (i, 0)),
            pl.BlockSpec((KB, 1), lambda i: (i, 0)),
            pl.BlockSpec((M, C), lambda i: (0, 0)),
            pl.BlockSpec((1, C), lambda i: (0, 0)),
            pl.BlockSpec((1, C), lambda i: (0, 0)),
            pl.BlockSpec((C, C), lambda i: (0, 0)),
            pl.BlockSpec((1, C), lambda i: (0, 0)),
            pl.BlockSpec((C, 1), lambda i: (0, 0)),
            pl.BlockSpec((1, 1), lambda i: (0, 0)),
        ],
        out_specs=(
            pl.BlockSpec((rows, C), lambda i: (i, 0)),
            pl.BlockSpec((1, 1, rows), lambda i: (i, 0, 0)),
        ),
        out_shape=(
            jax.ShapeDtypeStruct((N * M, C), jnp.float32),
            jax.ShapeDtypeStruct((NBLK, 1, rows), jnp.float32),
        ),
    )(s2, mask2, emb, ln1g, ln1b, W1, b1, W2, b2)


def _vfull(val, dtype=jnp.int32):
    return jnp.full((16,), val, dtype)


def _geom_sc_body(geom_hbm, mu_out, sig_out, rows_ref,
                  mu_stage, sig_stage, sem, *, n_child, m):
    nc = 2
    wid = lax.axis_index("s") * nc + lax.axis_index("c")
    base = wid * n_child
    n_par = n_child // m
    base_p = wid * n_par
    lane = lax.broadcasted_iota(jnp.int32, (16,), 0)

    pltpu.sync_copy(geom_hbm.at[pl.ds(base_p, n_par)], rows_ref)

    def mu_vec(v, o):
        child = lax.div(o, _vfull(3))
        comp = o - child * _vfull(3)
        par = lax.div(child, _vfull(m))
        val = plsc.load_gather(rows_ref, [par, comp])
        plsc.store_scatter(mu_stage, [child, comp], val)
        return o + _vfull(16)
    lax.fori_loop(0, (n_child * 3) // 16, mu_vec, lane)

    def sig_vec(v, o):
        child = lax.div(o, _vfull(9))
        col = o - child * _vfull(9)
        par = lax.div(child, _vfull(m))
        val = plsc.load_gather(rows_ref, [par, col + _vfull(3)])
        diag = ((col == _vfull(0)) | (col == _vfull(4))
                | (col == _vfull(8)))
        val = (val * _vfull(PHI ** -2, jnp.float32)
               + jnp.where(diag, _vfull(JITTER, jnp.float32),
                           _vfull(0.0, jnp.float32)))
        r = lax.div(col, _vfull(3))
        plsc.store_scatter(sig_stage, [child, r, col - r * _vfull(3)], val)
        return o + _vfull(16)
    lax.fori_loop(0, (n_child * 9) // 16, sig_vec, lane)

    pltpu.sync_copy(mu_stage, mu_out.at[pl.ds(base, n_child)])
    pltpu.sync_copy(sig_stage, sig_out.at[pl.ds(base, n_child)])


def _sc_call(geom16, *, n_child_total, m):
    n_child = n_child_total // 32
    mesh = plsc.VectorSubcoreMesh(core_axis_name="c", subcore_axis_name="s")
    body = functools.partial(_geom_sc_body, n_child=n_child, m=m)
    return pl.kernel(
        body,
        out_type=(
            jax.ShapeDtypeStruct((n_child_total, 3), jnp.float32),
            jax.ShapeDtypeStruct((n_child_total, 3, 3), jnp.float32),
        ),
        mesh=mesh,
        compiler_params=pltpu.CompilerParams(use_tc_tiling_on_sc=False,
                                             needs_layout_passes=False),
        scratch_types=[
            pltpu.VMEM((n_child // m, 16), jnp.float32),
            pltpu.VMEM((n_child, 3), jnp.float32),
            pltpu.VMEM((n_child, 3, 3), jnp.float32),
            pltpu.SemaphoreType.DMA,
        ],
    )(geom16)


@jax.jit
def kernel(s_parent, mu_p, Sigma_p, mask_parent, xi_noise, params):
    B, Kp, C = s_parent.shape
    M = M_MAX
    Kcand = Kp * M
    N = B * Kp

    s2 = s_parent.reshape(N, C)
    mask2 = mask_parent.reshape(N, 1)
    geom16 = jnp.concatenate(
        [mu_p.reshape(N, 3), Sigma_p.reshape(N, 9),
         jnp.zeros((N, 4), jnp.float32)], axis=-1)

    p = params
    ln1g = p['ln1_g'].reshape(1, C)
    ln1b = p['ln1_b'].reshape(1, C)
    b1 = p['b1'].reshape(1, C)
    b2 = p['b2'].reshape(1, 1)

    s_child, g = _tc_call(s2, mask2, p['embed_w'], ln1g, ln1b, p['W1'],
                          b1, p['W2'], b2, N=N, C=C, M=M, KB=128)
    mu_child, sig_child = _sc_call(geom16, n_child_total=N * M, m=M)

    s_child0 = s_child.reshape(B, Kcand, C)
    mu_child = mu_child.reshape(B, Kcand, 3)
    Sigma_child = sig_child.reshape(B, Kcand, 3, 3)
    g = g.reshape(B, Kcand)
    loss_count = jnp.zeros((), jnp.float32)
    return (s_child0, mu_child, Sigma_child, g, loss_count)


# speedup vs baseline: 2.1997x; 2.1997x over previous
import functools

import jax
import jax.numpy as jnp
from jax import lax
from jax.experimental import pallas as pl

M_MAX = 8
PHI = 1.6
JITTER = 1e-4


def _gate_up_kernel(s_ref, geom_ref, mask_ref, emb_ref, ln1g_ref, ln1b_ref,
                    w1_ref, b1_ref, w2_ref, b2_ref,
                    s_child_ref, geom_child_ref, g_ref, *, kb, m):
    s_blk = s_ref[...]
    C = s_blk.shape[1]
    rows = kb * m

    s_rep = jnp.broadcast_to(s_blk[:, None, :], (kb, m, C)).reshape(rows, C)
    e_rep = jnp.broadcast_to(emb_ref[...][None, :, :], (kb, m, C)).reshape(rows, C)

    gate_in = s_rep + e_rep
    mu = jnp.mean(gate_in, axis=-1, keepdims=True)
    var = jnp.mean(jnp.square(gate_in - mu), axis=-1, keepdims=True)
    h = (gate_in - mu) * lax.rsqrt(var + 1e-5)
    h = h * ln1g_ref[...] + ln1b_ref[...]

    h1 = jnp.dot(h, w1_ref[...], preferred_element_type=jnp.float32) + b1_ref[...]
    h1 = h1 * jax.nn.sigmoid(h1)
    bg = jnp.dot(h1, w2_ref[...], preferred_element_type=jnp.float32) + b2_ref[...]

    m_rep = jnp.broadcast_to(mask_ref[...][:, None, :], (kb, m, 1)).reshape(rows, 1)
    g = jax.nn.sigmoid(bg) * m_rep

    s_child_ref[...] = g * s_rep
    g_ref[...] = jnp.transpose(g, (1, 0))[None]

    geom_blk = geom_ref[...]
    geom_rep = jnp.broadcast_to(geom_blk[:, None, :], (kb, m, 12)).reshape(rows, 12)
    idx = lax.broadcasted_iota(jnp.int32, (1, 12), 1)
    scale = jnp.where(idx < 3, 1.0, PHI ** -2).astype(jnp.float32)
    shift = jnp.where((idx == 3) | (idx == 7) | (idx == 11),
                      JITTER, 0.0).astype(jnp.float32)
    geom_child_ref[...] = geom_rep * scale + shift


@jax.jit
def kernel(s_parent, mu_p, Sigma_p, mask_parent, xi_noise, params):
    B, Kp, C = s_parent.shape
    M = M_MAX
    Kcand = Kp * M
    N = B * Kp
    KB = 128
    NBLK = N // KB
    rows = KB * M

    s2 = s_parent.reshape(N, C)
    geom = jnp.concatenate(
        [mu_p.reshape(N, 3), Sigma_p.reshape(N, 9)], axis=-1)
    mask2 = mask_parent.reshape(N, 1)

    p = params
    emb = p['embed_w']
    ln1g = p['ln1_g'].reshape(1, C)
    ln1b = p['ln1_b'].reshape(1, C)
    b1 = p['b1'].reshape(1, C)
    b2 = p['b2'].reshape(1, 1)

    kfn = functools.partial(_gate_up_kernel, kb=KB, m=M)
    out_shapes = (
        jax.ShapeDtypeStruct((N * M, C), jnp.float32),
        jax.ShapeDtypeStruct((N * M, 12), jnp.float32),
        jax.ShapeDtypeStruct((NBLK, 1, rows), jnp.float32),
    )
    in_specs = [
        pl.BlockSpec((KB, C), lambda i: (i, 0)),
        pl.BlockSpec((KB, 12), lambda i: (i, 0)),
        pl.BlockSpec((KB, 1), lambda i: (i, 0)),
        pl.BlockSpec((M, C), lambda i: (0, 0)),
        pl.BlockSpec((1, C), lambda i: (0, 0)),
        pl.BlockSpec((1, C), lambda i: (0, 0)),
        pl.BlockSpec((C, C), lambda i: (0, 0)),
        pl.BlockSpec((1, C), lambda i: (0, 0)),
        pl.BlockSpec((C, 1), lambda i: (0, 0)),
        pl.BlockSpec((1, 1), lambda i: (0, 0)),
    ]
    out_specs = (
        pl.BlockSpec((rows, C), lambda i: (i, 0)),
        pl.BlockSpec((rows, 12), lambda i: (i, 0)),
        pl.BlockSpec((1, 1, rows), lambda i: (i, 0, 0)),
    )
    s_child, geom_child, g = pl.pallas_call(
        kfn,
        grid=(NBLK,),
        in_specs=in_specs,
        out_specs=out_specs,
        out_shape=out_shapes,
    )(s2, geom, mask2, emb, ln1g, ln1b, p['W1'], b1, p['W2'], b2)

    s_child0 = s_child.reshape(B, Kcand, C)
    geom_child = geom_child.reshape(B, Kcand, 12)
    mu_child = geom_child[..., :3]
    Sigma_child = geom_child[..., 3:].reshape(B, Kcand, 3, 3)
    g = g.reshape(B, Kcand)
    loss_count = jnp.zeros((), jnp.float32)
    return (s_child0, mu_child, Sigma_child, g, loss_count)


# speedup vs baseline: 2.2361x; 1.0166x over previous
import functools

import jax
import jax.numpy as jnp
from jax import lax
from jax.experimental import pallas as pl

M_MAX = 8
PHI = 1.6
JITTER = 1e-4


def _gate_up_kernel(s_ref, geom_ref, mask_ref, emb_ref, ln1g_ref, ln1b_ref,
                    w1_ref, b1_ref, w2_ref, b2_ref,
                    s_child_ref, geom_child_ref, g_ref, *, kb, m):
    s_blk = s_ref[...]
    C = s_blk.shape[1]
    rows = kb * m

    s_rep = jnp.broadcast_to(s_blk[:, None, :], (kb, m, C)).reshape(rows, C)
    e_rep = jnp.broadcast_to(emb_ref[...][None, :, :], (kb, m, C)).reshape(rows, C)

    gate_in = s_rep + e_rep
    mu = jnp.mean(gate_in, axis=-1, keepdims=True)
    var = jnp.mean(jnp.square(gate_in - mu), axis=-1, keepdims=True)
    h = (gate_in - mu) * lax.rsqrt(var + 1e-5)
    h = h * ln1g_ref[...] + ln1b_ref[...]

    h1 = jnp.dot(h, w1_ref[...], preferred_element_type=jnp.float32) + b1_ref[...]
    h1 = h1 * jax.nn.sigmoid(h1)
    bg = jnp.dot(h1, w2_ref[...], preferred_element_type=jnp.float32) + b2_ref[...]

    m_rep = jnp.broadcast_to(mask_ref[...][:, None, :], (kb, m, 1)).reshape(rows, 1)
    g = jax.nn.sigmoid(bg) * m_rep

    s_child_ref[...] = g * s_rep
    g_ref[...] = jnp.transpose(g, (1, 0))[None]

    geom_blk = geom_ref[...]
    geom_rep = jnp.broadcast_to(geom_blk[:, None, :], (kb, m, 12)).reshape(rows, 12)
    idx = lax.broadcasted_iota(jnp.int32, (1, 12), 1)
    scale = jnp.where(idx < 3, 1.0, PHI ** -2).astype(jnp.float32)
    shift = jnp.where((idx == 3) | (idx == 7) | (idx == 11),
                      JITTER, 0.0).astype(jnp.float32)
    geom_child_ref[...] = geom_rep * scale + shift


@jax.jit
def kernel(s_parent, mu_p, Sigma_p, mask_parent, xi_noise, params):
    B, Kp, C = s_parent.shape
    M = M_MAX
    Kcand = Kp * M
    N = B * Kp
    KB = 256
    NBLK = N // KB
    rows = KB * M

    s2 = s_parent.reshape(N, C)
    geom = jnp.concatenate(
        [mu_p.reshape(N, 3), Sigma_p.reshape(N, 9)], axis=-1)
    mask2 = mask_parent.reshape(N, 1)

    p = params
    emb = p['embed_w']
    ln1g = p['ln1_g'].reshape(1, C)
    ln1b = p['ln1_b'].reshape(1, C)
    b1 = p['b1'].reshape(1, C)
    b2 = p['b2'].reshape(1, 1)

    kfn = functools.partial(_gate_up_kernel, kb=KB, m=M)
    out_shapes = (
        jax.ShapeDtypeStruct((N * M, C), jnp.float32),
        jax.ShapeDtypeStruct((N * M, 12), jnp.float32),
        jax.ShapeDtypeStruct((NBLK, 1, rows), jnp.float32),
    )
    in_specs = [
        pl.BlockSpec((KB, C), lambda i: (i, 0)),
        pl.BlockSpec((KB, 12), lambda i: (i, 0)),
        pl.BlockSpec((KB, 1), lambda i: (i, 0)),
        pl.BlockSpec((M, C), lambda i: (0, 0)),
        pl.BlockSpec((1, C), lambda i: (0, 0)),
        pl.BlockSpec((1, C), lambda i: (0, 0)),
        pl.BlockSpec((C, C), lambda i: (0, 0)),
        pl.BlockSpec((1, C), lambda i: (0, 0)),
        pl.BlockSpec((C, 1), lambda i: (0, 0)),
        pl.BlockSpec((1, 1), lambda i: (0, 0)),
    ]
    out_specs = (
        pl.BlockSpec((rows, C), lambda i: (i, 0)),
        pl.BlockSpec((rows, 12), lambda i: (i, 0)),
        pl.BlockSpec((1, 1, rows), lambda i: (i, 0, 0)),
    )
    s_child, geom_child, g = pl.pallas_call(
        kfn,
        grid=(NBLK,),
        in_specs=in_specs,
        out_specs=out_specs,
        out_shape=out_shapes,
    )(s2, geom, mask2, emb, ln1g, ln1b, p['W1'], b1, p['W2'], b2)

    s_child0 = s_child.reshape(B, Kcand, C)
    geom_child = geom_child.reshape(B, Kcand, 12)
    mu_child = geom_child[..., :3]
    Sigma_child = geom_child[..., 3:].reshape(B, Kcand, 3, 3)
    g = g.reshape(B, Kcand)
    loss_count = jnp.zeros((), jnp.float32)
    return (s_child0, mu_child, Sigma_child, g, loss_count)
